# Initial kernel scaffold; baseline (speedup 1.0000x reference)
import jax
import jax.numpy as jnp
from jax.experimental import pallas as pl


def kernel(x, edge_index, edge_attr, emb_table, ex_W, ex_b, ex_lng, ex_lnb, ea_W1, ea_b1, ea_W2, ea_b2, ea_lng, ea_lnb, gc_W1, gc_b1, gc_W2, gc_b2, gc_xlng, gc_xlnb, gc_elng, gc_elnb, out_W, out_b):
    raise NotImplementedError("write your pallas kernel here")


# speedup vs baseline: 1.1139x; 1.1139x over previous
import functools

import jax
import jax.numpy as jnp
from jax import lax
from jax.experimental import pallas as pl
from jax.experimental.pallas import tpu as pltpu
from jax.experimental.pallas import tpu_sc as plsc

N = 10000
E = 320000
H = 128
DE = 16
NW = 32
NP = 10240
N_PER_W = NP // NW
E_PER_W = E // NW
GCH = 80
NCH_E = E_PER_W // GCH
SCH = 128
SEG_LD = 336
F32 = jnp.float32
I32 = jnp.int32


def _mesh():
    return plsc.VectorSubcoreMesh(core_axis_name="c", subcore_axis_name="s")


def _wid():
    return lax.axis_index("s") * 2 + lax.axis_index("c")


def _sc_embed(emb_table, x_p):
    @functools.partial(
        pl.kernel,
        out_type=jax.ShapeDtypeStruct((NP, H), F32),
        mesh=_mesh(),
        scratch_types=[pltpu.VMEM((64,), I32),
                       pltpu.VMEM((64, H), F32),
                       pltpu.SemaphoreType.DMA],
    )
    def k(emb_hbm, x_hbm, hx_hbm, nidx, nrows, sem):
        nbase = _wid() * N_PER_W

        def nbody(c, _):
            off = nbase + c * 64
            pltpu.sync_copy(x_hbm.at[pl.ds(off, 64)], nidx)
            pltpu.async_copy(emb_hbm.at[nidx], nrows, sem).wait()
            pltpu.sync_copy(nrows, hx_hbm.at[pl.ds(off, 64)])
            return 0

        lax.fori_loop(0, N_PER_W // 64, nbody, 0)

    return k(emb_table, x_p)


def _sc_gather(a, row, col):
    @functools.partial(
        pl.kernel,
        out_type=[jax.ShapeDtypeStruct((E, H), F32),
                  jax.ShapeDtypeStruct((E, H), F32)],
        mesh=_mesh(),
        scratch_types=[pltpu.VMEM((GCH,), I32),
                       pltpu.VMEM((GCH,), I32),
                       pltpu.VMEM((GCH, H), F32),
                       pltpu.VMEM((GCH, H), F32),
                       pltpu.SemaphoreType.DMA,
                       pltpu.SemaphoreType.DMA],
    )
    def k(a_hbm, r_hbm, c_hbm, gr_hbm, gc_hbm,
          ridx, cidx, rbuf, cbuf, sem1, sem2):
        base = _wid() * E_PER_W

        def body(ch, _):
            off = base + ch * GCH
            pltpu.sync_copy(r_hbm.at[pl.ds(off, GCH)], ridx)
            pltpu.sync_copy(c_hbm.at[pl.ds(off, GCH)], cidx)
            cp1 = pltpu.async_copy(a_hbm.at[ridx], rbuf, sem1)
            cp2 = pltpu.async_copy(a_hbm.at[cidx], cbuf, sem2)
            cp1.wait()
            cp2.wait()
            pltpu.sync_copy(rbuf, gr_hbm.at[pl.ds(off, GCH)])
            pltpu.sync_copy(cbuf, gc_hbm.at[pl.ds(off, GCH)])
            return 0

        lax.fori_loop(0, NCH_E, body, 0)

    return k(a, row, col)


def _sc_segsum(out_e, r_sp, perm_p, seg):
    @functools.partial(
        pl.kernel,
        out_type=jax.ShapeDtypeStruct((NP, H), F32),
        mesh=_mesh(),
        scratch_types=[pltpu.VMEM((SEG_LD,), I32),
                       pltpu.VMEM((N_PER_W + 1, H), F32),
                       pltpu.VMEM((SCH, H), F32),
                       pltpu.VMEM((SCH + 16,), I32),
                       pltpu.VMEM((SCH,), I32),
                       pltpu.SemaphoreType.DMA],
    )
    def k(oute_hbm, r_hbm, perm_hbm, seg_hbm, agg_hbm,
          segv, acc, buf, ridx, pidx, sem):
        w = _wid()
        nbase = w * N_PER_W
        pltpu.sync_copy(seg_hbm.at[pl.ds(nbase, SEG_LD)], segv)
        es = segv[pl.ds(0, 16)][0]
        ee = segv[pl.ds(N_PER_W, 16)][0]
        es8 = (es // 8) * 8
        nchunks = (ee - es8 + (SCH - 1)) // SCH

        def zbody(r, _):
            for j in range(H // 16):
                acc[r, pl.ds(j * 16, 16)] = jnp.zeros((16,), F32)
            return 0

        lax.fori_loop(0, N_PER_W + 1, zbody, 0)

        def cbody(ch, _):
            off = es8 + ch * SCH
            pltpu.sync_copy(r_hbm.at[pl.ds(off, SCH)],
                            ridx.at[pl.ds(0, SCH)])
            pltpu.sync_copy(perm_hbm.at[pl.ds(off, SCH)], pidx)
            pltpu.async_copy(oute_hbm.at[pidx], buf, sem).wait()

            def ebody(kk, _):
                ge = off + kk
                valid = jnp.logical_and(ge >= es, ge < ee)
                r_loc = jnp.where(valid, ridx[pl.ds(kk, 16)][0] - nbase,
                                  N_PER_W)
                for j in range(H // 16):
                    sl = pl.ds(j * 16, 16)
                    acc[r_loc, sl] = acc[r_loc, sl] + buf[kk, sl]
                return 0

            lax.fori_loop(0, SCH, ebody, 0)
            return 0

        lax.fori_loop(0, nchunks, cbody, 0)
        pltpu.sync_copy(acc.at[pl.ds(0, N_PER_W)],
                        agg_hbm.at[pl.ds(nbase, N_PER_W)])

    return k(out_e, r_sp, perm_p, seg)


def _ln(y, g, b):
    m = jnp.mean(y, -1, keepdims=True)
    v = jnp.mean((y - m) ** 2, -1, keepdims=True)
    return (y - m) * lax.rsqrt(v + 1e-5) * g + b


def _tc_node_prologue(hx, ex_W, ex_b, ex_lng, ex_lnb):
    BLK = 1024

    def body(hx_ref, w_ref, b_ref, g_ref, bb_ref, o_ref):
        h = jnp.maximum(hx_ref[...], 0.0)
        y = jnp.dot(h, w_ref[...], preferred_element_type=F32) + b_ref[...]
        o_ref[...] = _ln(y, g_ref[...], bb_ref[...])

    return pl.pallas_call(
        body,
        grid=(NP // BLK,),
        in_specs=[pl.BlockSpec((BLK, H), lambda i: (i, 0)),
                  pl.BlockSpec((H, H), lambda i: (0, 0)),
                  pl.BlockSpec((1, H), lambda i: (0, 0)),
                  pl.BlockSpec((1, H), lambda i: (0, 0)),
                  pl.BlockSpec((1, H), lambda i: (0, 0))],
        out_specs=pl.BlockSpec((BLK, H), lambda i: (i, 0)),
        out_shape=jax.ShapeDtypeStruct((NP, H), F32),
    )(hx, ex_W, ex_b.reshape(1, H), ex_lng.reshape(1, H), ex_lnb.reshape(1, H))


def _tc_edge_prologue(ea, W1, b1, W2, b2, lng, lnb):
    BLK = 512

    def body(e_ref, w1, b1r, w2, b2r, g, b, o_ref):
        t = jnp.dot(e_ref[...], w1[...], preferred_element_type=F32) + b1r[...]
        t = jnp.maximum(t, 0.0)
        y = jnp.dot(t, w2[...], preferred_element_type=F32) + b2r[...]
        o_ref[...] = _ln(y, g[...], b[...])

    return pl.pallas_call(
        body,
        grid=(E // BLK,),
        in_specs=[pl.BlockSpec((BLK, DE), lambda i: (i, 0)),
                  pl.BlockSpec((DE, H), lambda i: (0, 0)),
                  pl.BlockSpec((1, H), lambda i: (0, 0)),
                  pl.BlockSpec((H, H), lambda i: (0, 0)),
                  pl.BlockSpec((1, H), lambda i: (0, 0)),
                  pl.BlockSpec((1, H), lambda i: (0, 0)),
                  pl.BlockSpec((1, H), lambda i: (0, 0))],
        out_specs=pl.BlockSpec((BLK, H), lambda i: (i, 0)),
        out_shape=jax.ShapeDtypeStruct((E, H), F32),
    )(ea, W1, b1.reshape(1, H), W2, b2.reshape(1, H),
      lng.reshape(1, H), lnb.reshape(1, H))


def _tc_edge_mlp(gr, gc, be, W1r, W1c, W1e, b1, W2, b2, elng, elnb, want_next):
    BLK = 512
    H2 = 2 * H

    def body(gr_ref, gc_ref, be_ref, w1r, w1c, w1e, b1r, w2, b2r, g, b, *outs):
        t = (jnp.dot(gr_ref[...], w1r[...], preferred_element_type=F32)
             + jnp.dot(gc_ref[...], w1c[...], preferred_element_type=F32)
             + jnp.dot(be_ref[...], w1e[...], preferred_element_type=F32)
             + b1r[...])
        t = jnp.maximum(t, 0.0)
        o = jnp.dot(t, w2[...], preferred_element_type=F32) + b2r[...]
        outs[0][...] = o
        if want_next:
            outs[1][...] = jnp.maximum(be_ref[...] + _ln(o, g[...], b[...]),
                                       0.0)

    out_shapes = [jax.ShapeDtypeStruct((E, H), F32)]
    out_specs = [pl.BlockSpec((BLK, H), lambda i: (i, 0))]
    if want_next:
        out_shapes.append(jax.ShapeDtypeStruct((E, H), F32))
        out_specs.append(pl.BlockSpec((BLK, H), lambda i: (i, 0)))

    res = pl.pallas_call(
        body,
        grid=(E // BLK,),
        in_specs=[pl.BlockSpec((BLK, H), lambda i: (i, 0)),
                  pl.BlockSpec((BLK, H), lambda i: (i, 0)),
                  pl.BlockSpec((BLK, H), lambda i: (i, 0)),
                  pl.BlockSpec((H, H2), lambda i: (0, 0)),
                  pl.BlockSpec((H, H2), lambda i: (0, 0)),
                  pl.BlockSpec((H, H2), lambda i: (0, 0)),
                  pl.BlockSpec((1, H2), lambda i: (0, 0)),
                  pl.BlockSpec((H2, H), lambda i: (0, 0)),
                  pl.BlockSpec((1, H), lambda i: (0, 0)),
                  pl.BlockSpec((1, H), lambda i: (0, 0)),
                  pl.BlockSpec((1, H), lambda i: (0, 0))],
        out_specs=out_specs,
        out_shape=out_shapes,
    )(gr, gc, be, W1r, W1c, W1e, b1.reshape(1, H2), W2, b2.reshape(1, H),
      elng.reshape(1, H), elnb.reshape(1, H))
    return res if want_next else (res[0], None)


def _tc_node_update(a, agg, g, b, relu_out):
    BLK = 1024

    def body(a_ref, agg_ref, g_ref, b_ref, o_ref):
        y = a_ref[...] + _ln(agg_ref[...], g_ref[...], b_ref[...])
        o_ref[...] = jnp.maximum(y, 0.0) if relu_out else y

    return pl.pallas_call(
        body,
        grid=(NP // BLK,),
        in_specs=[pl.BlockSpec((BLK, H), lambda i: (i, 0)),
                  pl.BlockSpec((BLK, H), lambda i: (i, 0)),
                  pl.BlockSpec((1, H), lambda i: (0, 0)),
                  pl.BlockSpec((1, H), lambda i: (0, 0))],
        out_specs=pl.BlockSpec((BLK, H), lambda i: (i, 0)),
        out_shape=jax.ShapeDtypeStruct((NP, H), F32),
    )(a, agg, g.reshape(1, H), b.reshape(1, H))


def _tc_final(a, agg, g, b, out_W, out_b):
    BLK = 1024

    def body(a_ref, agg_ref, g_ref, b_ref, w_ref, ob_ref, o_ref):
        h4 = a_ref[...] + _ln(agg_ref[...], g_ref[...], b_ref[...])
        o_ref[...] = (jnp.dot(h4, w_ref[...], preferred_element_type=F32)
                      + ob_ref[...])

    return pl.pallas_call(
        body,
        grid=(NP // BLK,),
        in_specs=[pl.BlockSpec((BLK, H), lambda i: (i, 0)),
                  pl.BlockSpec((BLK, H), lambda i: (i, 0)),
                  pl.BlockSpec((1, H), lambda i: (0, 0)),
                  pl.BlockSpec((1, H), lambda i: (0, 0)),
                  pl.BlockSpec((H, H), lambda i: (0, 0)),
                  pl.BlockSpec((1, H), lambda i: (0, 0))],
        out_specs=pl.BlockSpec((BLK, H), lambda i: (i, 0)),
        out_shape=jax.ShapeDtypeStruct((NP, H), F32),
    )(a, agg, g.reshape(1, H), b.reshape(1, H), out_W, out_b.reshape(1, H))


def kernel(x, edge_index, edge_attr, emb_table, ex_W, ex_b, ex_lng, ex_lnb,
           ea_W1, ea_b1, ea_W2, ea_b2, ea_lng, ea_lnb,
           gc_W1, gc_b1, gc_W2, gc_b2, gc_xlng, gc_xlnb, gc_elng, gc_elnb,
           out_W, out_b):
    row, col = edge_index[0], edge_index[1]
    perm = jnp.argsort(row).astype(I32)
    r_s = row[perm]
    r_sp = jnp.concatenate([r_s, jnp.full((SCH,), NP - 1, I32)])
    perm_p = jnp.concatenate([perm, jnp.zeros((SCH,), I32)])
    x_p = jnp.concatenate([x, jnp.zeros((NP - N,), I32)])
    seg = jnp.searchsorted(r_s, jnp.arange(NP + 32, dtype=I32),
                           side="left").astype(I32)

    hx = _sc_embed(emb_table, x_p)
    a = _tc_node_prologue(hx, ex_W, ex_b, ex_lng, ex_lnb)
    b = _tc_edge_prologue(edge_attr, ea_W1, ea_b1, ea_W2, ea_b2,
                          ea_lng, ea_lnb)

    for i in range(4):
        gr, gc = _sc_gather(a, row, col)
        W1 = gc_W1[i]
        oute, bnext = _tc_edge_mlp(
            gr, gc, b, W1[0:H], W1[H:2 * H], W1[2 * H:3 * H],
            gc_b1[i], gc_W2[i], gc_b2[i], gc_elng[i], gc_elnb[i],
            want_next=(i < 3))
        agg = _sc_segsum(oute, r_sp, perm_p, seg)
        if i < 3:
            a = _tc_node_update(a, agg, gc_xlng[i], gc_xlnb[i], relu_out=True)
            b = bnext
        else:
            out = _tc_final(a, agg, gc_xlng[i], gc_xlnb[i], out_W, out_b)
    return out[:N]


# speedup vs baseline: 1.2247x; 1.0995x over previous
import functools

import jax
import jax.numpy as jnp
from jax import lax
from jax.experimental import pallas as pl
from jax.experimental.pallas import tpu as pltpu
from jax.experimental.pallas import tpu_sc as plsc

N = 10000
E = 320000
H = 128
DE = 16
NW = 32
NP = 10240
N_PER_W = NP // NW
E_PER_W = E // NW
GCH = 80
NCH_E = E_PER_W // GCH
SCH = 128
SEG_LD = 336
F32 = jnp.float32
I32 = jnp.int32


def _mesh():
    return plsc.VectorSubcoreMesh(core_axis_name="c", subcore_axis_name="s")


def _wid():
    return lax.axis_index("s") * 2 + lax.axis_index("c")


def _sc_embed(emb_table, x_p):
    @functools.partial(
        pl.kernel,
        out_type=jax.ShapeDtypeStruct((NP, H), F32),
        mesh=_mesh(),
        scratch_types=[pltpu.VMEM((64,), I32),
                       pltpu.VMEM((64, H), F32),
                       pltpu.SemaphoreType.DMA],
    )
    def k(emb_hbm, x_hbm, hx_hbm, nidx, nrows, sem):
        nbase = _wid() * N_PER_W

        def nbody(c, _):
            off = nbase + c * 64
            pltpu.sync_copy(x_hbm.at[pl.ds(off, 64)], nidx)
            pltpu.async_copy(emb_hbm.at[nidx], nrows, sem).wait()
            pltpu.sync_copy(nrows, hx_hbm.at[pl.ds(off, 64)])
            return 0

        lax.fori_loop(0, N_PER_W // 64, nbody, 0)

    return k(emb_table, x_p)


def _sc_gather(a, row, col):
    @functools.partial(
        pl.kernel,
        out_type=[jax.ShapeDtypeStruct((E, H), F32),
                  jax.ShapeDtypeStruct((E, H), F32)],
        mesh=_mesh(),
        scratch_types=[pltpu.VMEM((GCH,), I32), pltpu.VMEM((GCH,), I32),
                       pltpu.VMEM((GCH,), I32), pltpu.VMEM((GCH,), I32),
                       pltpu.VMEM((GCH, H), F32), pltpu.VMEM((GCH, H), F32),
                       pltpu.VMEM((GCH, H), F32), pltpu.VMEM((GCH, H), F32),
                       pltpu.SemaphoreType.DMA, pltpu.SemaphoreType.DMA,
                       pltpu.SemaphoreType.DMA, pltpu.SemaphoreType.DMA],
    )
    def k(a_hbm, r_hbm, c_hbm, gr_hbm, gc_hbm,
          ridx0, cidx0, ridx1, cidx1, rbuf0, cbuf0, rbuf1, cbuf1,
          sr0, sc0, sr1, sc1):
        base = _wid() * E_PER_W
        slots = ((ridx0, cidx0, rbuf0, cbuf0, sr0, sc0),
                 (ridx1, cidx1, rbuf1, cbuf1, sr1, sc1))

        def start(ch, q):
            ridx, cidx, rbuf, cbuf, sr, sc = slots[q]
            off = base + ch * GCH
            pltpu.sync_copy(r_hbm.at[pl.ds(off, GCH)], ridx)
            pltpu.sync_copy(c_hbm.at[pl.ds(off, GCH)], cidx)
            pltpu.async_copy(a_hbm.at[ridx], rbuf, sr)
            pltpu.async_copy(a_hbm.at[cidx], cbuf, sc)

        def finish(ch, q):
            ridx, cidx, rbuf, cbuf, sr, sc = slots[q]
            off = base + ch * GCH
            pltpu.make_async_copy(a_hbm.at[ridx], rbuf, sr).wait()
            pltpu.make_async_copy(a_hbm.at[cidx], cbuf, sc).wait()
            pltpu.sync_copy(rbuf, gr_hbm.at[pl.ds(off, GCH)])
            pltpu.sync_copy(cbuf, gc_hbm.at[pl.ds(off, GCH)])

        start(0, 0)

        def body(p, _):
            start(2 * p + 1, 1)
            finish(2 * p, 0)
            start(2 * p + 2, 0)
            finish(2 * p + 1, 1)
            return 0

        lax.fori_loop(0, (NCH_E - 1) // 2, body, 0)
        finish(NCH_E - 1, 0)

    return k(a, row, col)


def _sc_segsum(out_e, r_sp, perm_p, seg):
    @functools.partial(
        pl.kernel,
        out_type=jax.ShapeDtypeStruct((NP, H), F32),
        mesh=_mesh(),
        scratch_types=[pltpu.VMEM((SEG_LD,), I32),
                       pltpu.VMEM((N_PER_W + 1, H), F32),
                       pltpu.VMEM((SCH, H), F32),
                       pltpu.VMEM((SCH, H), F32),
                       pltpu.VMEM((SCH + 16,), I32),
                       pltpu.VMEM((SCH + 16,), I32),
                       pltpu.VMEM((SCH,), I32),
                       pltpu.VMEM((SCH,), I32),
                       pltpu.SemaphoreType.DMA,
                       pltpu.SemaphoreType.DMA],
    )
    def k(oute_hbm, r_hbm, perm_hbm, seg_hbm, agg_hbm,
          segv, acc, buf0, buf1, ridx0, ridx1, pidx0, pidx1, sg0, sg1):
        w = _wid()
        nbase = w * N_PER_W
        pltpu.sync_copy(seg_hbm.at[pl.ds(nbase, SEG_LD)], segv)
        es = segv[pl.ds(0, 16)][0]
        ee = segv[pl.ds(N_PER_W, 16)][0]
        es8 = (es // 8) * 8
        nchunks = (ee - es8 + (SCH - 1)) // SCH
        slots = ((buf0, ridx0, pidx0, sg0), (buf1, ridx1, pidx1, sg1))

        def zbody(r, _):
            for j in range(H // 16):
                acc[r, pl.ds(j * 16, 16)] = jnp.zeros((16,), F32)
            return 0

        lax.fori_loop(0, N_PER_W + 1, zbody, 0)

        def start(ch, q):
            buf, ridx, pidx, sg = slots[q]
            off = es8 + ch * SCH
            pltpu.sync_copy(r_hbm.at[pl.ds(off, SCH)],
                            ridx.at[pl.ds(0, SCH)])
            pltpu.sync_copy(perm_hbm.at[pl.ds(off, SCH)], pidx)
            pltpu.async_copy(oute_hbm.at[pidx], buf, sg)

        def accum(ch, q):
            buf, ridx, pidx, sg = slots[q]
            off = es8 + ch * SCH
            pltpu.make_async_copy(oute_hbm.at[pidx], buf, sg).wait()

            def ebody(kk, _):
                ge = off + kk
                valid = jnp.logical_and(ge >= es, ge < ee)
                r_loc = jnp.where(valid, ridx[pl.ds(kk, 16)][0] - nbase,
                                  N_PER_W)
                for j in range(H // 16):
                    sl = pl.ds(j * 16, 16)
                    acc[r_loc, sl] = acc[r_loc, sl] + buf[kk, sl]
                return 0

            lax.fori_loop(0, SCH, ebody, 0)

        @pl.when(nchunks > 0)
        def _():
            start(0, 0)

        def cbody(p, _):
            ch0 = 2 * p
            ch1 = ch0 + 1

            @pl.when(ch1 < nchunks)
            def _():
                start(ch1, 1)

            accum(ch0, 0)

            @pl.when(ch1 + 1 < nchunks)
            def _():
                start(ch1 + 1, 0)

            @pl.when(ch1 < nchunks)
            def _():
                accum(ch1, 1)

            return 0

        lax.fori_loop(0, (nchunks + 1) // 2, cbody, 0)
        pltpu.sync_copy(acc.at[pl.ds(0, N_PER_W)],
                        agg_hbm.at[pl.ds(nbase, N_PER_W)])

    return k(out_e, r_sp, perm_p, seg)


def _ln(y, g, b):
    m = jnp.mean(y, -1, keepdims=True)
    v = jnp.mean((y - m) ** 2, -1, keepdims=True)
    return (y - m) * lax.rsqrt(v + 1e-5) * g + b


def _tc_node_prologue(hx, ex_W, ex_b, ex_lng, ex_lnb):
    BLK = 1024

    def body(hx_ref, w_ref, b_ref, g_ref, bb_ref, o_ref):
        h = jnp.maximum(hx_ref[...], 0.0)
        y = jnp.dot(h, w_ref[...], preferred_element_type=F32) + b_ref[...]
        o_ref[...] = _ln(y, g_ref[...], bb_ref[...])

    return pl.pallas_call(
        body,
        grid=(NP // BLK,),
        in_specs=[pl.BlockSpec((BLK, H), lambda i: (i, 0)),
                  pl.BlockSpec((H, H), lambda i: (0, 0)),
                  pl.BlockSpec((1, H), lambda i: (0, 0)),
                  pl.BlockSpec((1, H), lambda i: (0, 0)),
                  pl.BlockSpec((1, H), lambda i: (0, 0))],
        out_specs=pl.BlockSpec((BLK, H), lambda i: (i, 0)),
        out_shape=jax.ShapeDtypeStruct((NP, H), F32),
    )(hx, ex_W, ex_b.reshape(1, H), ex_lng.reshape(1, H), ex_lnb.reshape(1, H))


def _tc_edge_prologue(ea, W1, b1, W2, b2, lng, lnb):
    BLK = 512

    def body(e_ref, w1, b1r, w2, b2r, g, b, o_ref):
        t = jnp.dot(e_ref[...], w1[...], preferred_element_type=F32) + b1r[...]
        t = jnp.maximum(t, 0.0)
        y = jnp.dot(t, w2[...], preferred_element_type=F32) + b2r[...]
        o_ref[...] = _ln(y, g[...], b[...])

    return pl.pallas_call(
        body,
        grid=(E // BLK,),
        in_specs=[pl.BlockSpec((BLK, DE), lambda i: (i, 0)),
                  pl.BlockSpec((DE, H), lambda i: (0, 0)),
                  pl.BlockSpec((1, H), lambda i: (0, 0)),
                  pl.BlockSpec((H, H), lambda i: (0, 0)),
                  pl.BlockSpec((1, H), lambda i: (0, 0)),
                  pl.BlockSpec((1, H), lambda i: (0, 0)),
                  pl.BlockSpec((1, H), lambda i: (0, 0))],
        out_specs=pl.BlockSpec((BLK, H), lambda i: (i, 0)),
        out_shape=jax.ShapeDtypeStruct((E, H), F32),
    )(ea, W1, b1.reshape(1, H), W2, b2.reshape(1, H),
      lng.reshape(1, H), lnb.reshape(1, H))


def _tc_edge_mlp(gr, gc, be, W1r, W1c, W1e, b1, W2, b2, elng, elnb, want_next):
    BLK = 512
    H2 = 2 * H

    def body(gr_ref, gc_ref, be_ref, w1r, w1c, w1e, b1r, w2, b2r, g, b, *outs):
        t = (jnp.dot(gr_ref[...], w1r[...], preferred_element_type=F32)
             + jnp.dot(gc_ref[...], w1c[...], preferred_element_type=F32)
             + jnp.dot(be_ref[...], w1e[...], preferred_element_type=F32)
             + b1r[...])
        t = jnp.maximum(t, 0.0)
        o = jnp.dot(t, w2[...], preferred_element_type=F32) + b2r[...]
        outs[0][...] = o
        if want_next:
            outs[1][...] = jnp.maximum(be_ref[...] + _ln(o, g[...], b[...]),
                                       0.0)

    out_shapes = [jax.ShapeDtypeStruct((E, H), F32)]
    out_specs = [pl.BlockSpec((BLK, H), lambda i: (i, 0))]
    if want_next:
        out_shapes.append(jax.ShapeDtypeStruct((E, H), F32))
        out_specs.append(pl.BlockSpec((BLK, H), lambda i: (i, 0)))

    res = pl.pallas_call(
        body,
        grid=(E // BLK,),
        in_specs=[pl.BlockSpec((BLK, H), lambda i: (i, 0)),
                  pl.BlockSpec((BLK, H), lambda i: (i, 0)),
                  pl.BlockSpec((BLK, H), lambda i: (i, 0)),
                  pl.BlockSpec((H, H2), lambda i: (0, 0)),
                  pl.BlockSpec((H, H2), lambda i: (0, 0)),
                  pl.BlockSpec((H, H2), lambda i: (0, 0)),
                  pl.BlockSpec((1, H2), lambda i: (0, 0)),
                  pl.BlockSpec((H2, H), lambda i: (0, 0)),
                  pl.BlockSpec((1, H), lambda i: (0, 0)),
                  pl.BlockSpec((1, H), lambda i: (0, 0)),
                  pl.BlockSpec((1, H), lambda i: (0, 0))],
        out_specs=out_specs,
        out_shape=out_shapes,
    )(gr, gc, be, W1r, W1c, W1e, b1.reshape(1, H2), W2, b2.reshape(1, H),
      elng.reshape(1, H), elnb.reshape(1, H))
    return res if want_next else (res[0], None)


def _tc_node_update(a, agg, g, b, relu_out):
    BLK = 1024

    def body(a_ref, agg_ref, g_ref, b_ref, o_ref):
        y = a_ref[...] + _ln(agg_ref[...], g_ref[...], b_ref[...])
        o_ref[...] = jnp.maximum(y, 0.0) if relu_out else y

    return pl.pallas_call(
        body,
        grid=(NP // BLK,),
        in_specs=[pl.BlockSpec((BLK, H), lambda i: (i, 0)),
                  pl.BlockSpec((BLK, H), lambda i: (i, 0)),
                  pl.BlockSpec((1, H), lambda i: (0, 0)),
                  pl.BlockSpec((1, H), lambda i: (0, 0))],
        out_specs=pl.BlockSpec((BLK, H), lambda i: (i, 0)),
        out_shape=jax.ShapeDtypeStruct((NP, H), F32),
    )(a, agg, g.reshape(1, H), b.reshape(1, H))


def _tc_final(a, agg, g, b, out_W, out_b):
    BLK = 1024

    def body(a_ref, agg_ref, g_ref, b_ref, w_ref, ob_ref, o_ref):
        h4 = a_ref[...] + _ln(agg_ref[...], g_ref[...], b_ref[...])
        o_ref[...] = (jnp.dot(h4, w_ref[...], preferred_element_type=F32)
                      + ob_ref[...])

    return pl.pallas_call(
        body,
        grid=(NP // BLK,),
        in_specs=[pl.BlockSpec((BLK, H), lambda i: (i, 0)),
                  pl.BlockSpec((BLK, H), lambda i: (i, 0)),
                  pl.BlockSpec((1, H), lambda i: (0, 0)),
                  pl.BlockSpec((1, H), lambda i: (0, 0)),
                  pl.BlockSpec((H, H), lambda i: (0, 0)),
                  pl.BlockSpec((1, H), lambda i: (0, 0))],
        out_specs=pl.BlockSpec((BLK, H), lambda i: (i, 0)),
        out_shape=jax.ShapeDtypeStruct((NP, H), F32),
    )(a, agg, g.reshape(1, H), b.reshape(1, H), out_W, out_b.reshape(1, H))


def kernel(x, edge_index, edge_attr, emb_table, ex_W, ex_b, ex_lng, ex_lnb,
           ea_W1, ea_b1, ea_W2, ea_b2, ea_lng, ea_lnb,
           gc_W1, gc_b1, gc_W2, gc_b2, gc_xlng, gc_xlnb, gc_elng, gc_elnb,
           out_W, out_b):
    row, col = edge_index[0], edge_index[1]
    perm = jnp.argsort(row).astype(I32)
    r_s = row[perm]
    r_sp = jnp.concatenate([r_s, jnp.full((SCH,), NP - 1, I32)])
    perm_p = jnp.concatenate([perm, jnp.zeros((SCH,), I32)])
    x_p = jnp.concatenate([x, jnp.zeros((NP - N,), I32)])
    seg = jnp.searchsorted(r_s, jnp.arange(NP + 32, dtype=I32),
                           side="left").astype(I32)

    hx = _sc_embed(emb_table, x_p)
    a = _tc_node_prologue(hx, ex_W, ex_b, ex_lng, ex_lnb)
    b = _tc_edge_prologue(edge_attr, ea_W1, ea_b1, ea_W2, ea_b2,
                          ea_lng, ea_lnb)

    for i in range(4):
        gr, gc = _sc_gather(a, row, col)
        W1 = gc_W1[i]
        oute, bnext = _tc_edge_mlp(
            gr, gc, b, W1[0:H], W1[H:2 * H], W1[2 * H:3 * H],
            gc_b1[i], gc_W2[i], gc_b2[i], gc_elng[i], gc_elnb[i],
            want_next=(i < 3))
        agg = _sc_segsum(oute, r_sp, perm_p, seg)
        if i < 3:
            a = _tc_node_update(a, agg, gc_xlng[i], gc_xlnb[i], relu_out=True)
            b = bnext
        else:
            out = _tc_final(a, agg, gc_xlng[i], gc_xlnb[i], out_W, out_b)
    return out[:N]
